# initial kernel scaffold (unmeasured)
import jax
import jax.numpy as jnp
from jax import lax
from jax.experimental import pallas as pl
from jax.experimental.pallas import tpu as pltpu

N_DEV = 4
M, K, N = 4096, 4096, 8192
CH = M // N_DEV
TILE = 512
_MESH = pl.DeviceIdType.MESH


def kernel(x, w_mat):
    partial = jnp.dot(x, w_mat, preferred_element_type=jnp.float32)

    def body(p_ref, out_ref, acc_ref, recv_ref, va, vb,
             local_sems, rs_send, rs_recv, ag_send, ag_recv):
        my = lax.axis_index("i")
        left = jnp.mod(my - 1, N_DEV)
        right = jnp.mod(my + 1, N_DEV)

        barrier = pltpu.get_barrier_semaphore()
        for nbr in (left, right):
            pl.semaphore_signal(barrier, inc=1, device_id=(nbr,),
                                device_id_type=_MESH)
        pl.semaphore_wait(barrier, 2)

        def add_chunk(c_recv, s, dst_row0, to_out, do_silu):
            for t in range(CH // TILE):
                cp_a = pltpu.make_async_copy(
                    p_ref.at[pl.ds(c_recv * CH + t * TILE, TILE), :],
                    va, local_sems.at[0])
                cp_b = pltpu.make_async_copy(
                    recv_ref.at[s, pl.ds(t * TILE, TILE), :],
                    vb, local_sems.at[1])
                cp_a.start()
                cp_b.start()
                cp_a.wait()
                cp_b.wait()
                y = va[...] + vb[...]
                if do_silu:
                    y = y * jax.nn.sigmoid(y)
                va[...] = y
                if to_out:
                    dst = out_ref.at[pl.ds(dst_row0 + t * TILE, TILE), :]
                else:
                    dst = acc_ref.at[s, pl.ds(t * TILE, TILE), :]
                cp_o = pltpu.make_async_copy(va, dst, local_sems.at[0])
                cp_o.start()
                cp_o.wait()

        for s in range(N_DEV - 1):
            c_send = jnp.mod(my - s, N_DEV)
            if s == 0:
                src = p_ref.at[pl.ds(c_send * CH, CH), :]
            else:
                src = acc_ref.at[s - 1]
            rdma = pltpu.make_async_remote_copy(
                src_ref=src,
                dst_ref=recv_ref.at[s],
                send_sem=rs_send.at[s],
                recv_sem=rs_recv.at[s],
                device_id=(right,),
                device_id_type=_MESH,
            )
            rdma.start()
            rdma.wait()
            c_recv = jnp.mod(my - s - 1, N_DEV)
            if s < N_DEV - 2:
                add_chunk(c_recv, s, None, to_out=False, do_silu=False)
            else:
                add_chunk(c_recv, s, c_recv * CH, to_out=True, do_silu=True)

        for s in range(N_DEV - 1):
            c_send = jnp.mod(my + 1 - s, N_DEV)
            rows = pl.ds(c_send * CH, CH)
            rdma = pltpu.make_async_remote_copy(
                src_ref=out_ref.at[rows, :],
                dst_ref=out_ref.at[rows, :],
                send_sem=ag_send.at[s],
                recv_sem=ag_recv.at[s],
                device_id=(right,),
                device_id_type=_MESH,
            )
            rdma.start()
            rdma.wait()

    return pl.pallas_call(
        body,
        out_shape=jax.ShapeDtypeStruct((M, N), jnp.float32),
        in_specs=[pl.BlockSpec(memory_space=pltpu.HBM)],
        out_specs=pl.BlockSpec(memory_space=pltpu.HBM),
        scratch_shapes=[
            pltpu.HBM((N_DEV - 2, CH, N), jnp.float32),
            pltpu.HBM((N_DEV - 1, CH, N), jnp.float32),
            pltpu.VMEM((TILE, N), jnp.float32),
            pltpu.VMEM((TILE, N), jnp.float32),
            pltpu.SemaphoreType.DMA((2,)),
            pltpu.SemaphoreType.DMA((N_DEV - 1,)),
            pltpu.SemaphoreType.DMA((N_DEV - 1,)),
            pltpu.SemaphoreType.DMA((N_DEV - 1,)),
            pltpu.SemaphoreType.DMA((N_DEV - 1,)),
        ],
        compiler_params=pltpu.CompilerParams(collective_id=0),
    )(partial)


# baseline (device time: 2454244 ns/iter reference)
import jax
import jax.numpy as jnp
from jax import lax
from jax.experimental import pallas as pl
from jax.experimental.pallas import tpu as pltpu

N_DEV = 4
M, K, N = 4096, 4096, 8192
CH = M // N_DEV
TILE = 512
_MESH = pl.DeviceIdType.MESH


def kernel(x, w_mat):
    partial = jnp.dot(x, w_mat, preferred_element_type=jnp.float32)

    def body(p_ref, out_ref, acc_ref, recv_ref, va, vb,
             local_sems, rs_send, rs_recv, ag_send, ag_recv):
        my = lax.axis_index("i")
        left = jnp.mod(my - 1, N_DEV)
        right = jnp.mod(my + 1, N_DEV)

        barrier = pltpu.get_barrier_semaphore()
        for nbr in (left, right):
            pl.semaphore_signal(barrier, inc=1, device_id=(nbr,),
                                device_id_type=_MESH)
        pl.semaphore_wait(barrier, 2)

        def add_chunk(c_recv, s, dst_row0, to_out, do_silu):
            for t in range(CH // TILE):
                cp_a = pltpu.make_async_copy(
                    p_ref.at[pl.ds(c_recv * CH + t * TILE, TILE), :],
                    va, local_sems.at[0])
                cp_b = pltpu.make_async_copy(
                    recv_ref.at[s, pl.ds(t * TILE, TILE), :],
                    vb, local_sems.at[1])
                cp_a.start()
                cp_b.start()
                cp_a.wait()
                cp_b.wait()
                y = va[...] + vb[...]
                if do_silu:
                    y = y * jax.nn.sigmoid(y)
                va[...] = y
                if to_out:
                    dst = out_ref.at[pl.ds(dst_row0 + t * TILE, TILE), :]
                else:
                    dst = acc_ref.at[s, pl.ds(t * TILE, TILE), :]
                cp_o = pltpu.make_async_copy(va, dst, local_sems.at[0])
                cp_o.start()
                cp_o.wait()

        for s in range(N_DEV - 1):
            c_send = jnp.mod(my - s, N_DEV)
            if s == 0:
                src = p_ref.at[pl.ds(c_send * CH, CH), :]
            else:
                src = acc_ref.at[s - 1]
            rdma = pltpu.make_async_remote_copy(
                src_ref=src,
                dst_ref=recv_ref.at[s],
                send_sem=rs_send.at[s],
                recv_sem=rs_recv.at[s],
                device_id=(right,),
                device_id_type=_MESH,
            )
            rdma.start()
            rdma.wait()
            c_recv = jnp.mod(my - s - 1, N_DEV)
            if s < N_DEV - 2:
                add_chunk(c_recv, s, None, to_out=False, do_silu=False)
            else:
                add_chunk(c_recv, s, c_recv * CH, to_out=True, do_silu=True)

        for s in range(N_DEV - 1):
            c_send = jnp.mod(my + 1 - s, N_DEV)
            rows = pl.ds(c_send * CH, CH)
            rdma = pltpu.make_async_remote_copy(
                src_ref=out_ref.at[rows, :],
                dst_ref=out_ref.at[rows, :],
                send_sem=ag_send.at[s],
                recv_sem=ag_recv.at[s],
                device_id=(right,),
                device_id_type=_MESH,
            )
            rdma.start()
            rdma.wait()

    out, _, _ = pl.pallas_call(
        body,
        out_shape=(
            jax.ShapeDtypeStruct((M, N), jnp.float32),
            jax.ShapeDtypeStruct((N_DEV - 2, CH, N), jnp.float32),
            jax.ShapeDtypeStruct((N_DEV - 1, CH, N), jnp.float32),
        ),
        in_specs=[pl.BlockSpec(memory_space=pltpu.HBM)],
        out_specs=(
            pl.BlockSpec(memory_space=pltpu.HBM),
            pl.BlockSpec(memory_space=pltpu.HBM),
            pl.BlockSpec(memory_space=pltpu.HBM),
        ),
        scratch_shapes=[
            pltpu.VMEM((TILE, N), jnp.float32),
            pltpu.VMEM((TILE, N), jnp.float32),
            pltpu.SemaphoreType.DMA((2,)),
            pltpu.SemaphoreType.DMA((N_DEV - 1,)),
            pltpu.SemaphoreType.DMA((N_DEV - 1,)),
            pltpu.SemaphoreType.DMA((N_DEV - 1,)),
            pltpu.SemaphoreType.DMA((N_DEV - 1,)),
        ],
        compiler_params=pltpu.CompilerParams(collective_id=0),
    )(partial)
    return out


# device time: 1390094 ns/iter; 1.7655x vs baseline; 1.7655x over previous
import jax
import jax.numpy as jnp
from jax import lax
from jax.experimental import pallas as pl
from jax.experimental.pallas import tpu as pltpu

N_DEV = 4
M, K, N = 4096, 4096, 8192
CH = M // N_DEV
N2 = N // 2
TILE = 512
_MESH = pl.DeviceIdType.MESH


def kernel(x, w_mat):
    partial = jnp.dot(x, w_mat, preferred_element_type=jnp.float32)

    def body(p_ref, out_ref,
             accR_ref, recvR_ref, accL_ref, recvL_ref,
             va, vb, local_sems,
             rsR_send, rsR_recv, agR_send, agR_recv,
             rsL_send, rsL_recv, agL_send, agL_recv):
        my = lax.axis_index("i")
        left = jnp.mod(my - 1, N_DEV)
        right = jnp.mod(my + 1, N_DEV)

        barrier = pltpu.get_barrier_semaphore()
        for nbr in (left, right):
            pl.semaphore_signal(barrier, inc=1, device_id=(nbr,),
                                device_id_type=_MESH)
        pl.semaphore_wait(barrier, 2)

        def add_chunk(c_recv, s, recv_ref, acc_ref, col0, final):
            for t in range(CH // TILE):
                r0 = c_recv * CH + t * TILE
                cp_a = pltpu.make_async_copy(
                    p_ref.at[pl.ds(r0, TILE), pl.ds(col0, N2)],
                    va, local_sems.at[0])
                cp_b = pltpu.make_async_copy(
                    recv_ref.at[s, pl.ds(t * TILE, TILE), :],
                    vb, local_sems.at[1])
                cp_a.start()
                cp_b.start()
                cp_a.wait()
                cp_b.wait()
                y = va[...] + vb[...]
                if final:
                    y = y * jax.nn.sigmoid(y)
                    dst = out_ref.at[pl.ds(r0, TILE), pl.ds(col0, N2)]
                else:
                    dst = acc_ref.at[s, pl.ds(t * TILE, TILE), :]
                va[...] = y
                cp_o = pltpu.make_async_copy(va, dst, local_sems.at[0])
                cp_o.start()
                cp_o.wait()

        for s in range(N_DEV - 1):
            cR = jnp.mod(my - s, N_DEV)
            cL = jnp.mod(my + s, N_DEV)
            if s == 0:
                srcR = p_ref.at[pl.ds(cR * CH, CH), pl.ds(0, N2)]
                srcL = p_ref.at[pl.ds(cL * CH, CH), pl.ds(N2, N2)]
            else:
                srcR = accR_ref.at[s - 1]
                srcL = accL_ref.at[s - 1]
            rdmaR = pltpu.make_async_remote_copy(
                src_ref=srcR, dst_ref=recvR_ref.at[s],
                send_sem=rsR_send.at[s], recv_sem=rsR_recv.at[s],
                device_id=(right,), device_id_type=_MESH)
            rdmaL = pltpu.make_async_remote_copy(
                src_ref=srcL, dst_ref=recvL_ref.at[s],
                send_sem=rsL_send.at[s], recv_sem=rsL_recv.at[s],
                device_id=(left,), device_id_type=_MESH)
            rdmaR.start()
            rdmaL.start()
            rdmaR.wait()
            rdmaL.wait()
            final = s == N_DEV - 2
            add_chunk(jnp.mod(my - s - 1, N_DEV), s, recvR_ref, accR_ref,
                      0, final)
            add_chunk(jnp.mod(my + s + 1, N_DEV), s, recvL_ref, accL_ref,
                      N2, final)

        for s in range(N_DEV - 1):
            cR = jnp.mod(my + 1 - s, N_DEV)
            cL = jnp.mod(my - 1 + s, N_DEV)
            rowsR = pl.ds(cR * CH, CH)
            rowsL = pl.ds(cL * CH, CH)
            rdmaR = pltpu.make_async_remote_copy(
                src_ref=out_ref.at[rowsR, pl.ds(0, N2)],
                dst_ref=out_ref.at[rowsR, pl.ds(0, N2)],
                send_sem=agR_send.at[s], recv_sem=agR_recv.at[s],
                device_id=(right,), device_id_type=_MESH)
            rdmaL = pltpu.make_async_remote_copy(
                src_ref=out_ref.at[rowsL, pl.ds(N2, N2)],
                dst_ref=out_ref.at[rowsL, pl.ds(N2, N2)],
                send_sem=agL_send.at[s], recv_sem=agL_recv.at[s],
                device_id=(left,), device_id_type=_MESH)
            rdmaR.start()
            rdmaL.start()
            rdmaR.wait()
            rdmaL.wait()

    out, _, _, _, _ = pl.pallas_call(
        body,
        out_shape=(
            jax.ShapeDtypeStruct((M, N), jnp.float32),
            jax.ShapeDtypeStruct((N_DEV - 2, CH, N2), jnp.float32),
            jax.ShapeDtypeStruct((N_DEV - 1, CH, N2), jnp.float32),
            jax.ShapeDtypeStruct((N_DEV - 2, CH, N2), jnp.float32),
            jax.ShapeDtypeStruct((N_DEV - 1, CH, N2), jnp.float32),
        ),
        in_specs=[pl.BlockSpec(memory_space=pltpu.HBM)],
        out_specs=tuple(pl.BlockSpec(memory_space=pltpu.HBM)
                        for _ in range(5)),
        scratch_shapes=[
            pltpu.VMEM((TILE, N2), jnp.float32),
            pltpu.VMEM((TILE, N2), jnp.float32),
            pltpu.SemaphoreType.DMA((2,)),
            pltpu.SemaphoreType.DMA((N_DEV - 1,)),
            pltpu.SemaphoreType.DMA((N_DEV - 1,)),
            pltpu.SemaphoreType.DMA((N_DEV - 1,)),
            pltpu.SemaphoreType.DMA((N_DEV - 1,)),
            pltpu.SemaphoreType.DMA((N_DEV - 1,)),
            pltpu.SemaphoreType.DMA((N_DEV - 1,)),
            pltpu.SemaphoreType.DMA((N_DEV - 1,)),
            pltpu.SemaphoreType.DMA((N_DEV - 1,)),
        ],
        compiler_params=pltpu.CompilerParams(collective_id=0),
    )(partial)
    return out


# device time: 1284110 ns/iter; 1.9112x vs baseline; 1.0825x over previous
import jax
import jax.numpy as jnp
from jax import lax
from jax.experimental import pallas as pl
from jax.experimental.pallas import tpu as pltpu

N_DEV = 4
M, K, N = 4096, 4096, 8192
CH = M // N_DEV
N2 = N // 2
TILE = 512
T = CH // TILE
_MESH = pl.DeviceIdType.MESH


def kernel(x, w_mat):
    partial = jnp.dot(x, w_mat, preferred_element_type=jnp.float32)

    def body(p_ref, out_ref,
             accR_ref, recvR_ref, accL_ref, recvL_ref,
             va, vb, local_sems,
             rsR_send, rsR_recv, agR_send, agR_recv,
             rsL_send, rsL_recv, agL_send, agL_recv):
        my = lax.axis_index("i")
        left = jnp.mod(my - 1, N_DEV)
        right = jnp.mod(my + 1, N_DEV)

        barrier = pltpu.get_barrier_semaphore()
        for nbr in (left, right):
            pl.semaphore_signal(barrier, inc=1, device_id=(nbr,),
                                device_id_type=_MESH)
        pl.semaphore_wait(barrier, 2)

        sends = []

        def rs_rdma(s, t, src, recv_ref, sems_send, sems_recv, dst_dev):
            rdma = pltpu.make_async_remote_copy(
                src_ref=src,
                dst_ref=recv_ref.at[s, pl.ds(t * TILE, TILE), :],
                send_sem=sems_send.at[s, t], recv_sem=sems_recv.at[s, t],
                device_id=(dst_dev,), device_id_type=_MESH)
            return rdma

        for t in range(T):
            cR = my
            cL = my
            sR = rs_rdma(0, t,
                         p_ref.at[pl.ds(cR * CH + t * TILE, TILE),
                                  pl.ds(0, N2)],
                         recvR_ref, rsR_send, rsR_recv, right)
            sL = rs_rdma(0, t,
                         p_ref.at[pl.ds(cL * CH + t * TILE, TILE),
                                  pl.ds(N2, N2)],
                         recvL_ref, rsL_send, rsL_recv, left)
            sR.start()
            sL.start()
            sends += [sR, sL]

        def process_tile(s, t, c_recv, recv_ref, acc_ref, col0,
                         sems_send, sems_recv, dst_dev):
            rx = rs_rdma(s, t, recv_ref.at[s, pl.ds(t * TILE, TILE), :],
                         recv_ref, sems_send, sems_recv, dst_dev)
            rx.wait_recv()
            r0 = c_recv * CH + t * TILE
            cp_a = pltpu.make_async_copy(
                p_ref.at[pl.ds(r0, TILE), pl.ds(col0, N2)],
                va, local_sems.at[0])
            cp_b = pltpu.make_async_copy(
                recv_ref.at[s, pl.ds(t * TILE, TILE), :],
                vb, local_sems.at[1])
            cp_a.start()
            cp_b.start()
            cp_a.wait()
            cp_b.wait()
            y = va[...] + vb[...]
            if s == N_DEV - 2:
                y = y * jax.nn.sigmoid(y)
                dst = out_ref.at[pl.ds(r0, TILE), pl.ds(col0, N2)]
            else:
                dst = acc_ref.at[s, pl.ds(t * TILE, TILE), :]
            va[...] = y
            cp_o = pltpu.make_async_copy(va, dst, local_sems.at[0])
            cp_o.start()
            cp_o.wait()
            if s < N_DEV - 2:
                fwd = rs_rdma(s + 1, t, acc_ref.at[s, pl.ds(t * TILE, TILE), :],
                              recv_ref, sems_send, sems_recv, dst_dev)
                fwd.start()
                sends.append(fwd)

        for s in range(N_DEV - 1):
            for t in range(T):
                process_tile(s, t, jnp.mod(my - s - 1, N_DEV),
                             recvR_ref, accR_ref, 0, rsR_send, rsR_recv,
                             right)
                process_tile(s, t, jnp.mod(my + s + 1, N_DEV),
                             recvL_ref, accL_ref, N2, rsL_send, rsL_recv,
                             left)

        for s in range(N_DEV - 1):
            cR = jnp.mod(my + 1 - s, N_DEV)
            cL = jnp.mod(my - 1 + s, N_DEV)
            rowsR = pl.ds(cR * CH, CH)
            rowsL = pl.ds(cL * CH, CH)
            rdmaR = pltpu.make_async_remote_copy(
                src_ref=out_ref.at[rowsR, pl.ds(0, N2)],
                dst_ref=out_ref.at[rowsR, pl.ds(0, N2)],
                send_sem=agR_send.at[s], recv_sem=agR_recv.at[s],
                device_id=(right,), device_id_type=_MESH)
            rdmaL = pltpu.make_async_remote_copy(
                src_ref=out_ref.at[rowsL, pl.ds(N2, N2)],
                dst_ref=out_ref.at[rowsL, pl.ds(N2, N2)],
                send_sem=agL_send.at[s], recv_sem=agL_recv.at[s],
                device_id=(left,), device_id_type=_MESH)
            rdmaR.start()
            rdmaL.start()
            rdmaR.wait()
            rdmaL.wait()

        for s_ in sends:
            s_.wait_send()

    out, _, _, _, _ = pl.pallas_call(
        body,
        out_shape=(
            jax.ShapeDtypeStruct((M, N), jnp.float32),
            jax.ShapeDtypeStruct((N_DEV - 2, CH, N2), jnp.float32),
            jax.ShapeDtypeStruct((N_DEV - 1, CH, N2), jnp.float32),
            jax.ShapeDtypeStruct((N_DEV - 2, CH, N2), jnp.float32),
            jax.ShapeDtypeStruct((N_DEV - 1, CH, N2), jnp.float32),
        ),
        in_specs=[pl.BlockSpec(memory_space=pltpu.HBM)],
        out_specs=tuple(pl.BlockSpec(memory_space=pltpu.HBM)
                        for _ in range(5)),
        scratch_shapes=[
            pltpu.VMEM((TILE, N2), jnp.float32),
            pltpu.VMEM((TILE, N2), jnp.float32),
            pltpu.SemaphoreType.DMA((2,)),
            pltpu.SemaphoreType.DMA((N_DEV - 1, T)),
            pltpu.SemaphoreType.DMA((N_DEV - 1, T)),
            pltpu.SemaphoreType.DMA((N_DEV - 1,)),
            pltpu.SemaphoreType.DMA((N_DEV - 1,)),
            pltpu.SemaphoreType.DMA((N_DEV - 1, T)),
            pltpu.SemaphoreType.DMA((N_DEV - 1, T)),
            pltpu.SemaphoreType.DMA((N_DEV - 1,)),
            pltpu.SemaphoreType.DMA((N_DEV - 1,)),
        ],
        compiler_params=pltpu.CompilerParams(collective_id=0),
    )(partial)
    return out
